# Initial kernel scaffold; baseline (speedup 1.0000x reference)
import jax
import jax.numpy as jnp
from jax.experimental import pallas as pl


def kernel(x, spatial_pos, W1, b1, W2, b2, local_num_frames, is_image):
    raise NotImplementedError("write your pallas kernel here")


# speedup vs baseline: 1.9345x; 1.9345x over previous
import functools

import jax
import jax.numpy as jnp
from jax import lax
from jax.experimental import pallas as pl
from jax.experimental.pallas import tpu as pltpu

P0 = 729
C = 1152
CE = C + 128
HEADS = 16
DIM = 72
TGT = 128

ROUNDS = ((729, 365, 364, 364), (365, 183, 182, 182), (183, 92, 91, 55))

_f32 = jnp.float32


def _mm(a, b, trans_b=False, hi=False):
    dims = (((1,), (1 if trans_b else 0,)), ((), ()))
    prec = lax.Precision.HIGHEST if hi else None
    return lax.dot_general(a, b, dims, precision=prec,
                           preferred_element_type=_f32)


def _merge_round(s, na, nb, r):
    p = na + nb
    sz = s[:, C:C + 1]
    x = s[:, :C] / sz
    m = x[:, 0:DIM]
    for h in range(1, HEADS):
        m = m + x[:, DIM * h:DIM * (h + 1)]
    m = m * (1.0 / HEADS)
    m = m / jnp.sqrt(jnp.sum(m * m, axis=1, keepdims=True))
    ia = lax.broadcasted_iota(jnp.int32, (na, p), 0)
    qa = lax.broadcasted_iota(jnp.int32, (na, p), 1)
    Ea = (qa == 2 * ia).astype(_f32)
    ib = lax.broadcasted_iota(jnp.int32, (nb, p), 0)
    qb = lax.broadcasted_iota(jnp.int32, (nb, p), 1)
    Eb = (qb == 2 * ib + 1).astype(_f32)
    a = _mm(Ea, m, hi=True)
    bm = _mm(Eb, m, hi=True)
    scores = _mm(a, bm, trans_b=True)
    node_max = jnp.max(scores, axis=1, keepdims=True)
    jj = lax.broadcasted_iota(jnp.int32, (na, nb), 1)
    node_idx = jnp.min(jnp.where(scores == node_max, jj, nb),
                       axis=1, keepdims=True)
    vrow = jnp.transpose(node_max)
    ii = lax.broadcasted_iota(jnp.int32, (na, na), 0)
    jj2 = lax.broadcasted_iota(jnp.int32, (na, na), 1)
    beats = (vrow > node_max) | ((vrow == node_max) & (jj2 < ii))
    rank = jnp.sum(beats.astype(_f32), axis=1, keepdims=True)
    n_unm = na - r
    tgt = jnp.where(rank < r, n_unm + node_idx.astype(_f32), rank - r)
    p_out = n_unm + nb
    c_even = _mm(jnp.transpose(tgt), Ea, hi=True)
    q_io = lax.broadcasted_iota(jnp.int32, (1, p), 1)
    c = jnp.where(q_io % 2 == 1,
                  (n_unm + (q_io - 1) // 2).astype(_f32), c_even)
    k_io = lax.broadcasted_iota(jnp.int32, (p_out, p), 0).astype(_f32)
    Gfull = (k_io == c).astype(_f32)
    return _mm(Gfull, s, hi=True)


def _merge_body(x_ref, pos_ref, out_ref):
    x = x_ref[0] + pos_ref[0]
    s = jnp.concatenate([x, jnp.ones((P0, 128), _f32)], axis=1)
    for (_, na, nb, r) in ROUNDS:
        s = _merge_round(s, na, nb, r)
    out_ref[0] = s[:, :C] / s[:, C:C + 1]


def _mlp_body(xm_ref, w1_ref, b1_ref, w2_ref, b2_ref, out_ref, h_ref):
    n = pl.program_id(1)

    @pl.when(n == 0)
    def _():
        h = _mm(xm_ref[...], w1_ref[...]) + b1_ref[...]
        h_ref[...] = 0.5 * h * (1.0 + lax.erf(h * (2.0 ** -0.5)))

    out_ref[...] = _mm(h_ref[...], w2_ref[...]) + b2_ref[...]


def kernel(x, spatial_pos, W1, b1, W2, b2, local_num_frames=1, is_image=True):
    B = x.shape[0]
    merged = pl.pallas_call(
        _merge_body,
        grid=(B,),
        in_specs=[
            pl.BlockSpec((1, P0, C), lambda b: (b, 0, 0)),
            pl.BlockSpec((1, P0, C), lambda b: (0, 0, 0)),
        ],
        out_specs=pl.BlockSpec((1, TGT, C), lambda b: (b, 0, 0)),
        out_shape=jax.ShapeDtypeStruct((B, TGT, C), _f32),
    )(x, spatial_pos)

    M = B * TGT
    BM, BN = 512, 256
    H = W1.shape[1]
    N = W2.shape[1]
    out = pl.pallas_call(
        _mlp_body,
        grid=(M // BM, N // BN),
        in_specs=[
            pl.BlockSpec((BM, C), lambda m, n: (m, 0)),
            pl.BlockSpec((C, H), lambda m, n: (0, 0)),
            pl.BlockSpec((1, H), lambda m, n: (0, 0)),
            pl.BlockSpec((H, BN), lambda m, n: (0, n)),
            pl.BlockSpec((1, BN), lambda m, n: (0, n)),
        ],
        out_specs=pl.BlockSpec((BM, BN), lambda m, n: (m, n)),
        out_shape=jax.ShapeDtypeStruct((M, N), _f32),
        scratch_shapes=[pltpu.VMEM((BM, H), _f32)],
    )(merged.reshape(M, C), W1, b1.reshape(1, H), W2, b2.reshape(1, N))
    return out.reshape(B, TGT, N)
